# baseline (speedup 1.0000x reference)
import functools
import jax
import jax.numpy as jnp
from jax import lax
from jax.experimental import pallas as pl
from jax.experimental.pallas import tpu as pltpu
from jax.experimental.pallas import tpu_sc as plsc

_BLK = 2048


def _sc_gather_body(ts_hbm, w_hbm, row_hbm, ts_v, row_v):
    wid = lax.axis_index("s") * 2 + lax.axis_index("c")

    @pl.when(wid == 0)
    def _():
        pltpu.sync_copy(ts_hbm, ts_v)
        t = ts_v[...][0]
        pltpu.sync_copy(w_hbm.at[pl.ds(t, 1)], row_v)
        pltpu.sync_copy(row_v, row_hbm)


def _sc_gather(ts, W):
    D = W.shape[1]
    mesh = plsc.VectorSubcoreMesh(core_axis_name="c", subcore_axis_name="s")
    f = functools.partial(
        pl.kernel,
        mesh=mesh,
        out_type=jax.ShapeDtypeStruct((1, D), jnp.float32),
        scratch_types=[
            pltpu.VMEM((16,), jnp.int32),
            pltpu.VMEM((1, D), jnp.float32),
        ],
    )(_sc_gather_body)
    return f(ts, W)


def _tc_body(x_ref, row_ref, o_ref):
    o_ref[...] = x_ref[...] + row_ref[0, :][None, :]


def kernel(x, timestep, W):
    B, D = x.shape
    ts = jnp.full((16,), timestep, dtype=jnp.int32)
    row = _sc_gather(ts, W)
    return pl.pallas_call(
        _tc_body,
        grid=(B // _BLK,),
        in_specs=[
            pl.BlockSpec((_BLK, D), lambda i: (i, 0)),
            pl.BlockSpec((1, D), lambda i: (0, 0)),
        ],
        out_specs=pl.BlockSpec((_BLK, D), lambda i: (i, 0)),
        out_shape=jax.ShapeDtypeStruct((B, D), x.dtype),
    )(x, row)


# speedup vs baseline: 1.0223x; 1.0223x over previous
import functools
import jax
import jax.numpy as jnp
from jax import lax
from jax.experimental import pallas as pl
from jax.experimental.pallas import tpu as pltpu
from jax.experimental.pallas import tpu_sc as plsc

_BLK = 2048


def _scs_gather_body(ts_hbm, w_hbm, row_hbm, ts_s, row_spmem):
    cid = lax.axis_index("c")

    @pl.when(cid == 0)
    def _():
        pltpu.sync_copy(ts_hbm, ts_s)
        t = ts_s[0]
        pltpu.sync_copy(w_hbm.at[pl.ds(t, 1)], row_spmem)
        pltpu.sync_copy(row_spmem, row_hbm)


def _sc_gather(ts, W):
    D = W.shape[1]
    mesh = plsc.ScalarSubcoreMesh(axis_name="c")
    f = functools.partial(
        pl.kernel,
        mesh=mesh,
        out_type=jax.ShapeDtypeStruct((1, D), jnp.float32),
        scratch_types=[
            pltpu.SMEM((16,), jnp.int32),
            pltpu.VMEM_SHARED((1, D), jnp.float32),
        ],
    )(_scs_gather_body)
    return f(ts, W)


def _tc_body(x_ref, row_ref, o_ref):
    o_ref[...] = x_ref[...] + row_ref[0, :][None, :]


def kernel(x, timestep, W):
    B, D = x.shape
    ts = jnp.full((16,), timestep, dtype=jnp.int32)
    row = _sc_gather(ts, W)
    return pl.pallas_call(
        _tc_body,
        grid=(B // _BLK,),
        in_specs=[
            pl.BlockSpec((_BLK, D), lambda i: (i, 0)),
            pl.BlockSpec((1, D), lambda i: (0, 0)),
        ],
        out_specs=pl.BlockSpec((_BLK, D), lambda i: (i, 0)),
        out_shape=jax.ShapeDtypeStruct((B, D), x.dtype),
    )(x, row)


# speedup vs baseline: 1.0241x; 1.0017x over previous
import functools
import jax
import jax.numpy as jnp
from jax import lax
from jax.experimental import pallas as pl
from jax.experimental.pallas import tpu as pltpu
from jax.experimental.pallas import tpu_sc as plsc

_BLK = 2048


def _scs_gather_body(ts_hbm, w_hbm, row_hbm, ts_s, row_spmem):
    cid = lax.axis_index("c")

    @pl.when(cid == 0)
    def _():
        pltpu.sync_copy(ts_hbm, ts_s)
        t = ts_s[0]
        pltpu.sync_copy(w_hbm.at[pl.ds(t, 1)], row_hbm)


def _sc_gather(ts, W):
    D = W.shape[1]
    mesh = plsc.ScalarSubcoreMesh(axis_name="c")
    f = functools.partial(
        pl.kernel,
        mesh=mesh,
        out_type=jax.ShapeDtypeStruct((1, D), jnp.float32),
        scratch_types=[
            pltpu.SMEM((16,), jnp.int32),
            pltpu.VMEM_SHARED((1, D), jnp.float32),
        ],
    )(_scs_gather_body)
    return f(ts, W)


def _tc_body(x_ref, row_ref, o_ref):
    o_ref[...] = x_ref[...] + row_ref[0, :][None, :]


def kernel(x, timestep, W):
    B, D = x.shape
    ts = jnp.full((16,), timestep, dtype=jnp.int32)
    row = _sc_gather(ts, W)
    return pl.pallas_call(
        _tc_body,
        grid=(B // _BLK,),
        in_specs=[
            pl.BlockSpec((_BLK, D), lambda i: (i, 0)),
            pl.BlockSpec((1, D), lambda i: (0, 0)),
        ],
        out_specs=pl.BlockSpec((_BLK, D), lambda i: (i, 0)),
        out_shape=jax.ShapeDtypeStruct((B, D), x.dtype),
    )(x, row)


# speedup vs baseline: 1.0262x; 1.0021x over previous
import functools
import jax
import jax.numpy as jnp
from jax import lax
from jax.experimental import pallas as pl
from jax.experimental.pallas import tpu as pltpu
from jax.experimental.pallas import tpu_sc as plsc

_BLK = 2048


def _scs_gather_body(ts_hbm, w_hbm, row_hbm, ts_s):
    cid = lax.axis_index("c")

    @pl.when(cid == 0)
    def _():
        pltpu.sync_copy(ts_hbm, ts_s)
        t = ts_s[0]
        pltpu.sync_copy(w_hbm.at[pl.ds(t, 1)], row_hbm)


def _sc_gather(ts, W):
    D = W.shape[1]
    mesh = plsc.ScalarSubcoreMesh(axis_name="c")
    f = functools.partial(
        pl.kernel,
        mesh=mesh,
        out_type=jax.ShapeDtypeStruct((1, D), jnp.float32),
        scratch_types=[
            pltpu.SMEM((16,), jnp.int32),
        ],
    )(_scs_gather_body)
    return f(ts, W)


def _tc_body(x_ref, row_ref, o_ref):
    o_ref[...] = x_ref[...] + row_ref[0, :][None, :]


def kernel(x, timestep, W):
    B, D = x.shape
    ts = jnp.full((16,), timestep, dtype=jnp.int32)
    row = _sc_gather(ts, W)
    return pl.pallas_call(
        _tc_body,
        grid=(B // _BLK,),
        in_specs=[
            pl.BlockSpec((_BLK, D), lambda i: (i, 0)),
            pl.BlockSpec((1, D), lambda i: (0, 0)),
        ],
        out_specs=pl.BlockSpec((_BLK, D), lambda i: (i, 0)),
        out_shape=jax.ShapeDtypeStruct((B, D), x.dtype),
    )(x, row)
